# baseline (speedup 1.0000x reference)
import functools

import jax
import jax.numpy as jnp
from jax import lax
from jax.experimental import pallas as pl
from jax.experimental.pallas import tpu as pltpu
from jax.experimental.pallas import tpu_sc as plsc

HID = 64
B = 128
NW = 32
NBUF = 3

N_DEV = 100000
N_IP = 100000
N_USER = 100000
N_NOTE = 150000
N_EV = 50000
N_TOTAL = 500000


def _cdiv(a, b):
    return (a + b - 1) // b


def _pad8(n):
    return _cdiv(n, 8) * 8


C_DEV = _pad8(_cdiv(N_DEV, B))
C_IP = _pad8(_cdiv(N_IP, B))
C_USER = _pad8(_cdiv(N_USER, B))
C_NOTE = _pad8(_cdiv(N_NOTE, B))
C_EV = _pad8(_cdiv(N_EV, B))

CPW_DEV = _cdiv(C_DEV, NW)
CPW_IP = _cdiv(C_IP, NW)
CPW_USER = _cdiv(C_USER, NW)
CPW_NOTE = _cdiv(C_NOTE, NW)
CPW_EV = _cdiv(C_EV, NW)


def _stage_rows(cpw):
    return (_cdiv(cpw, 8) + 1) * 8


S_DEV = _stage_rows(CPW_DEV)
S_IP = _stage_rows(CPW_IP)
S_USER = _stage_rows(CPW_USER)
S_NOTE = _stage_rows(CPW_NOTE)
S_EV = _stage_rows(CPW_EV)


def _tables_body(de0, de1, de2, wdev, ee0, ee1, wev, ntt,
                 t0p, t1, t2, te0p, te1):
    wd = wdev[...]
    we = wev[...]
    n = ntt[...]
    t0p[...] = jnp.dot(de0[...], wd[0:16, :],
                       preferred_element_type=jnp.float32) + n[0:1, :]
    t1[...] = jnp.dot(de1[...], wd[16:24, :],
                      preferred_element_type=jnp.float32)
    t2[...] = jnp.dot(de2[...], wd[24:40, :],
                      preferred_element_type=jnp.float32)
    te0p[...] = jnp.dot(ee0[...], we[0:16, :],
                        preferred_element_type=jnp.float32) + n[4:5, :]
    te1[...] = jnp.dot(ee1[...], we[16:24, :],
                       preferred_element_type=jnp.float32)


def _tc_tables(de0, de1, de2, wdev, ee0, ee1, wev, ntt):
    return pl.pallas_call(
        _tables_body,
        out_shape=[
            jax.ShapeDtypeStruct((1000, HID), jnp.float32),
            jax.ShapeDtypeStruct((50, HID), jnp.float32),
            jax.ShapeDtypeStruct((20000, HID), jnp.float32),
            jax.ShapeDtypeStruct((500, HID), jnp.float32),
            jax.ShapeDtypeStruct((100, HID), jnp.float32),
        ],
    )(de0, de1, de2, wdev, ee0, ee1, wev, ntt)


_MESH = plsc.VectorSubcoreMesh(
    core_axis_name="c", subcore_axis_name="s", num_cores=2, num_subcores=16)


def _sc_body(d_nid, d_i0, d_i1, d_i2, i_nid, u_nid, n_nid, e_nid, e_i0, e_i1,
             t0p, t1, t2, te0p, te1, ud0, ud1, wu, bu, ntt, out,
             st_nid, st_a, st_b, st_c,
             acc0, acc1, acc2, bb0, bb1, bb2, cc0, cc1, cc2,
             cbuf, udv0, udv1, wuv, buv, nttv,
             gs0, gs1, gs2, ss0, ss1, ss2):
    wid = lax.axis_index("s") * 2 + lax.axis_index("c")
    accs = (acc0, acc1, acc2)
    bbs = (bb0, bb1, bb2)
    ccs = (cc0, cc1, cc2)
    gsems = (gs0, gs1, gs2)
    ssems = (ss0, ss1, ss2)

    def ranges(cpw, s, nc, rot):
        ws = lax.rem(wid + rot, NW)
        c_lo = ws * cpw
        cl = jnp.minimum((c_lo // 8) * 8, nc - s)
        return c_lo, pl.multiple_of(cl, 8)

    def run_pipelined(cpw, nc, c_lo, cl, mk_gathers, compute=None):
        ngrp = _cdiv(cpw, NBUF)

        def pred(s):
            return (s >= 0) & (s < cpw) & ((c_lo + s) < nc)

        def mk_scatter(r, b):
            return pltpu.make_async_copy(accs[b], out.at[st_nid.at[r]],
                                         ssems[b])

        def g_issue(s, b):
            @pl.when(pred(s))
            def _():
                c = c_lo + s
                for d in mk_gathers(c - cl, c, b):
                    d.start()

        def g_wait(s, b):
            @pl.when(pred(s))
            def _():
                c = c_lo + s
                for d in mk_gathers(c - cl, c, b):
                    d.wait()

        def s_issue(s, b):
            @pl.when(pred(s))
            def _():
                mk_scatter(c_lo + s - cl, b).start()

        def s_wait(s, b):
            @pl.when(pred(s))
            def _():
                mk_scatter(c_lo + s - cl, b).wait()

        g_issue(jnp.int32(0), 0)
        g_issue(jnp.int32(1), 1)

        def grp(g, _):
            s0 = g * NBUF
            for bi in range(NBUF):
                s = s0 + bi
                g_wait(s, bi)
                if compute is not None:
                    @pl.when(pred(s))
                    def _(s=s, bi=bi):
                        compute(s, bi)
                s_issue(s, bi)
                s_wait(s - 1, (bi - 1) % NBUF)
                g_issue(s + 2, (bi + 2) % NBUF)
            return 0
        lax.fori_loop(0, ngrp, grp, 0)
        s_wait(ngrp * NBUF - 1, (NBUF - 1) % NBUF)

    def run_const(cpw, nc, c_lo, cl):
        def issue(t, _):
            c = c_lo + t

            @pl.when(c < nc)
            def _():
                pltpu.async_copy(cbuf, out.at[st_nid.at[c - cl]], ss0)
            return 0
        lax.fori_loop(0, cpw, issue, 0)

        def drain(t, _):
            c = c_lo + t

            @pl.when(c < nc)
            def _():
                pltpu.make_async_copy(cbuf, out.at[st_nid.at[c - cl]],
                                      ss0).wait()
            return 0
        lax.fori_loop(0, cpw, drain, 0)

    def fill_cbuf(type_row):
        @plsc.parallel_loop(0, B, unroll=4)
        def _(j):
            for k in range(4):
                sl = pl.ds(k * 16, 16)
                cbuf[j, sl] = nttv[type_row, sl]

    pltpu.sync_copy(ntt, nttv)
    pltpu.sync_copy(wu, wuv)
    pltpu.sync_copy(bu, buv)

    _scope_dev = jax.named_scope("seg_dev"); _scope_dev.__enter__()
    c_lo, cl = ranges(CPW_DEV, S_DEV, C_DEV, 0)
    pltpu.sync_copy(d_nid.at[pl.ds(cl, S_DEV)], st_nid.at[pl.ds(0, S_DEV)])
    pltpu.sync_copy(d_i0.at[pl.ds(cl, S_DEV)], st_a)
    pltpu.sync_copy(d_i1.at[pl.ds(cl, S_DEV)], st_b)
    pltpu.sync_copy(d_i2.at[pl.ds(cl, S_DEV)], st_c)

    def dev_gathers(r, c, b):
        return [pltpu.make_async_copy(t0p.at[st_a.at[r]], accs[b], gsems[b]),
                pltpu.make_async_copy(t1.at[st_b.at[r]], bbs[b], gsems[b]),
                pltpu.make_async_copy(t2.at[st_c.at[r]], ccs[b], gsems[b])]

    def dev_compute(s, b):
        a, bbuf, cbuf2 = accs[b], bbs[b], ccs[b]

        @plsc.parallel_loop(0, B, unroll=4)
        def _(j):
            for k in range(4):
                sl = pl.ds(k * 16, 16)
                plsc.addupdate(a.at[j, sl], bbuf[j, sl] + cbuf2[j, sl])

    run_pipelined(CPW_DEV, C_DEV, c_lo, cl, dev_gathers, dev_compute)

    _scope_dev.__exit__(None, None, None)
    _scope_ip = jax.named_scope("seg_ip"); _scope_ip.__enter__()
    c_lo, cl = ranges(CPW_IP, S_IP, C_IP, 7)
    fill_cbuf(1)
    pltpu.sync_copy(i_nid.at[pl.ds(cl, S_IP)], st_nid.at[pl.ds(0, S_IP)])
    run_const(CPW_IP, C_IP, c_lo, cl)

    _scope_ip.__exit__(None, None, None)
    _scope_us = jax.named_scope("seg_user"); _scope_us.__enter__()
    c_lo, cl = ranges(CPW_USER, S_USER, C_USER, 13)
    pltpu.sync_copy(u_nid.at[pl.ds(cl, S_USER)], st_nid.at[pl.ds(0, S_USER)])
    pltpu.sync_copy(ud0.at[pl.ds(cl * B, S_USER * B)], udv0)
    pltpu.sync_copy(ud1.at[pl.ds(cl * B, S_USER * B)], udv1)

    def user_compute(s, b):
        a = accs[b]
        base = (c_lo + s - cl) * B

        @plsc.parallel_loop(0, B // 16, unroll=2)
        def _(m):
            u0v = udv0[pl.ds(base + m * 16, 16)]
            u1v = udv1[pl.ds(base + m * 16, 16)]
            for jj in range(16):
                j = m * 16 + jj
                for k in range(4):
                    sl = pl.ds(k * 16, 16)
                    a[j, sl] = (u0v[jj] * wuv[0, sl]
                                + u1v[jj] * wuv[1, sl] + buv[0, sl])

    run_pipelined(CPW_USER, C_USER, c_lo, cl, lambda r, c, b: [],
                  user_compute)

    _scope_us.__exit__(None, None, None)
    _scope_nt = jax.named_scope("seg_note"); _scope_nt.__enter__()
    c_lo, cl = ranges(CPW_NOTE, S_NOTE, C_NOTE, 19)
    fill_cbuf(3)
    pltpu.sync_copy(n_nid.at[pl.ds(cl, S_NOTE)], st_nid)
    run_const(CPW_NOTE, C_NOTE, c_lo, cl)

    _scope_nt.__exit__(None, None, None)
    _scope_ev = jax.named_scope("seg_ev"); _scope_ev.__enter__()
    c_lo, cl = ranges(CPW_EV, S_EV, C_EV, 26)
    pltpu.sync_copy(e_nid.at[pl.ds(cl, S_EV)], st_nid.at[pl.ds(0, S_EV)])
    pltpu.sync_copy(e_i0.at[pl.ds(cl, S_EV)], st_a.at[pl.ds(0, S_EV)])
    pltpu.sync_copy(e_i1.at[pl.ds(cl, S_EV)], st_b.at[pl.ds(0, S_EV)])

    def ev_gathers(r, c, b):
        return [pltpu.make_async_copy(te0p.at[st_a.at[r]], accs[b], gsems[b]),
                pltpu.make_async_copy(te1.at[st_b.at[r]], bbs[b], gsems[b])]

    def ev_compute(s, b):
        a, bbuf = accs[b], bbs[b]

        @plsc.parallel_loop(0, B, unroll=4)
        def _(j):
            for k in range(4):
                sl = pl.ds(k * 16, 16)
                plsc.addupdate(a.at[j, sl], bbuf[j, sl])

    run_pipelined(CPW_EV, C_EV, c_lo, cl, ev_gathers, ev_compute)
    _scope_ev.__exit__(None, None, None)


N_PAD = 3907 * B

_sc_scatter = functools.partial(
    pl.kernel,
    out_type=jax.ShapeDtypeStruct((N_PAD, HID), jnp.float32),
    mesh=_MESH,
    scratch_types=[
        pltpu.VMEM((S_NOTE, B), jnp.int32),
        pltpu.VMEM((S_DEV, B), jnp.int32),
        pltpu.VMEM((S_DEV, B), jnp.int32),
        pltpu.VMEM((S_DEV, B), jnp.int32),
        pltpu.VMEM((B, HID), jnp.float32),
        pltpu.VMEM((B, HID), jnp.float32),
        pltpu.VMEM((B, HID), jnp.float32),
        pltpu.VMEM((B, HID), jnp.float32),
        pltpu.VMEM((B, HID), jnp.float32),
        pltpu.VMEM((B, HID), jnp.float32),
        pltpu.VMEM((B, HID), jnp.float32),
        pltpu.VMEM((B, HID), jnp.float32),
        pltpu.VMEM((B, HID), jnp.float32),
        pltpu.VMEM((B, HID), jnp.float32),
        pltpu.VMEM((S_USER * B,), jnp.float32),
        pltpu.VMEM((S_USER * B,), jnp.float32),
        pltpu.VMEM((2, HID), jnp.float32),
        pltpu.VMEM((1, HID), jnp.float32),
        pltpu.VMEM((5, HID), jnp.float32),
        pltpu.SemaphoreType.DMA,
        pltpu.SemaphoreType.DMA,
        pltpu.SemaphoreType.DMA,
        pltpu.SemaphoreType.DMA,
        pltpu.SemaphoreType.DMA,
        pltpu.SemaphoreType.DMA,
    ],
    compiler_params=pltpu.CompilerParams(use_tc_tiling_on_sc=False),
)(_sc_body)


NCH = N_PAD // B
CPW_T = _cdiv(NCH, NW)


def _tr_body(src, t4, sb0, sb1, sb2, db0, db1, db2,
             rs0, rs1, rs2, ws0, ws1, ws2):
    wid = lax.axis_index("s") * 2 + lax.axis_index("c")
    sbs = (sb0, sb1, sb2)
    dbs = (db0, db1, db2)
    rsems = (rs0, rs1, rs2)
    wsems = (ws0, ws1, ws2)
    c_lo = wid * CPW_T
    base_iota = lax.iota(jnp.int32, 16)

    def pred(s):
        return (s >= 0) & (s < CPW_T) & ((c_lo + s) < NCH)

    def mk_read(c, b):
        return pltpu.make_async_copy(src.at[pl.ds(c * B, B)], sbs[b],
                                     rsems[b])

    def mk_write(c, b):
        return pltpu.make_async_copy(dbs[b], t4.at[:, c], wsems[b])

    def r_issue(s, b):
        @pl.when(pred(s))
        def _():
            mk_read(c_lo + s, b).start()

    def r_wait(s, b):
        @pl.when(pred(s))
        def _():
            mk_read(c_lo + s, b).wait()

    def w_issue(s, b):
        @pl.when(pred(s))
        def _():
            mk_write(c_lo + s, b).start()

    def w_wait(s, b):
        @pl.when(pred(s))
        def _():
            mk_write(c_lo + s, b).wait()

    def transpose(b):
        sb, db = sbs[b], dbs[b]

        @plsc.parallel_loop(0, 8, unroll=1)
        def _(i):
            for s8 in range(8):
                colv = base_iota * 0 + (i * 8 + s8)
                for m in range(8):
                    g = plsc.load_gather(sb, [base_iota + m * 16, colv])
                    db[i, s8, pl.ds(m * 16, 16)] = g

    r_issue(jnp.int32(0), 0)
    r_issue(jnp.int32(1), 1)

    def grp(g, _):
        s0 = g * NBUF
        for bi in range(NBUF):
            s = s0 + bi
            r_wait(s, bi)

            @pl.when(pred(s))
            def _(bi=bi):
                transpose(bi)
            w_issue(s, bi)
            w_wait(s - 1, (bi - 1) % NBUF)
            r_issue(s + 2, (bi + 2) % NBUF)
        return 0
    lax.fori_loop(0, _cdiv(CPW_T, NBUF), grp, 0)
    w_wait(_cdiv(CPW_T, NBUF) * NBUF - 1, (NBUF - 1) % NBUF)


_sc_transpose = functools.partial(
    pl.kernel,
    out_type=jax.ShapeDtypeStruct((8, NCH, 8, B), jnp.float32),
    mesh=_MESH,
    scratch_types=[
        pltpu.VMEM((B, HID), jnp.float32),
        pltpu.VMEM((B, HID), jnp.float32),
        pltpu.VMEM((B, HID), jnp.float32),
        pltpu.VMEM((8, 8, B), jnp.float32),
        pltpu.VMEM((8, 8, B), jnp.float32),
        pltpu.VMEM((8, 8, B), jnp.float32),
        pltpu.SemaphoreType.DMA,
        pltpu.SemaphoreType.DMA,
        pltpu.SemaphoreType.DMA,
        pltpu.SemaphoreType.DMA,
        pltpu.SemaphoreType.DMA,
        pltpu.SemaphoreType.DMA,
    ],
    compiler_params=pltpu.CompilerParams(use_tc_tiling_on_sc=False,
                                         needs_layout_passes=False),
)(_tr_body)


def _pad_chunks(x, nc):
    return jnp.pad(x, (0, nc * B - x.shape[0]), mode='edge').reshape(nc, B)


def kernel(device_node_id, ip_node_id, user_node_id, note_node_id,
           event_node_id, device_cat, event_cat, user_dense,
           node_type_table, dev_emb0, dev_emb1, dev_emb2, W_dev,
           ev_emb0, ev_emb1, W_ev, W_user, b_user):
    i32 = jnp.int32
    d_nid = _pad_chunks(device_node_id.astype(i32), C_DEV)
    d_i0 = _pad_chunks(device_cat[:, 0].astype(i32), C_DEV)
    d_i1 = _pad_chunks(device_cat[:, 1].astype(i32), C_DEV)
    d_i2 = _pad_chunks(device_cat[:, 2].astype(i32), C_DEV)
    i_nid = _pad_chunks(ip_node_id.astype(i32), C_IP)
    u_nid = _pad_chunks(user_node_id.astype(i32), C_USER)
    n_nid = _pad_chunks(note_node_id.astype(i32), C_NOTE)
    e_nid = _pad_chunks(event_node_id.astype(i32), C_EV)
    e_i0 = _pad_chunks(event_cat[:, 0].astype(i32), C_EV)
    e_i1 = _pad_chunks(event_cat[:, 1].astype(i32), C_EV)

    t0p, t1, t2, te0p, te1 = _tc_tables(
        dev_emb0, dev_emb1, dev_emb2, W_dev, ev_emb0, ev_emb1, W_ev,
        node_type_table)

    npad = C_USER * B - N_USER
    ud0 = jnp.pad(user_dense[:, 0], (0, npad), mode='edge')
    ud1 = jnp.pad(user_dense[:, 1], (0, npad), mode='edge')

    dense = _sc_scatter(d_nid, d_i0, d_i1, d_i2, i_nid, u_nid, n_nid,
                        e_nid, e_i0, e_i1, t0p, t1, t2, te0p, te1,
                        ud0, ud1, W_user, b_user.reshape(1, HID),
                        node_type_table)
    t4 = _sc_transpose(dense)
    return t4.transpose(1, 3, 0, 2).reshape(N_PAD, HID)[:N_TOTAL]


# speedup vs baseline: 1.0442x; 1.0442x over previous
import functools

import jax
import jax.numpy as jnp
from jax import lax
from jax.experimental import pallas as pl
from jax.experimental.pallas import tpu as pltpu
from jax.experimental.pallas import tpu_sc as plsc

HID = 64
B = 128
NW = 32
NBUF = 3

N_DEV = 100000
N_IP = 100000
N_USER = 100000
N_NOTE = 150000
N_EV = 50000
N_TOTAL = 500000


def _cdiv(a, b):
    return (a + b - 1) // b


def _pad8(n):
    return _cdiv(n, 8) * 8


C_DEV = _pad8(_cdiv(N_DEV, B))
C_IP = _pad8(_cdiv(N_IP, B))
C_USER = _pad8(_cdiv(N_USER, B))
C_NOTE = _pad8(_cdiv(N_NOTE, B))
C_EV = _pad8(_cdiv(N_EV, B))

CPW_DEV = _cdiv(C_DEV, NW)
CPW_IP = _cdiv(C_IP, NW)
CPW_USER = _cdiv(C_USER, NW)
CPW_NOTE = _cdiv(C_NOTE, NW)
CPW_EV = _cdiv(C_EV, NW)


def _stage_rows(cpw):
    return (_cdiv(cpw, 8) + 1) * 8


S_DEV = _stage_rows(CPW_DEV)
S_IP = _stage_rows(CPW_IP)
S_USER = _stage_rows(CPW_USER)
S_NOTE = _stage_rows(CPW_NOTE)
S_EV = _stage_rows(CPW_EV)


def _tables_body(de0, de1, de2, wdev, ee0, ee1, wev, ntt,
                 t0p, t1, t2, te0p, te1):
    wd = wdev[...]
    we = wev[...]
    n = ntt[...]
    t0p[...] = jnp.dot(de0[...], wd[0:16, :],
                       preferred_element_type=jnp.float32) + n[0:1, :]
    t1[...] = jnp.dot(de1[...], wd[16:24, :],
                      preferred_element_type=jnp.float32)
    t2[...] = jnp.dot(de2[...], wd[24:40, :],
                      preferred_element_type=jnp.float32)
    te0p[...] = jnp.dot(ee0[...], we[0:16, :],
                        preferred_element_type=jnp.float32) + n[4:5, :]
    te1[...] = jnp.dot(ee1[...], we[16:24, :],
                       preferred_element_type=jnp.float32)


def _tc_tables(de0, de1, de2, wdev, ee0, ee1, wev, ntt):
    return pl.pallas_call(
        _tables_body,
        out_shape=[
            jax.ShapeDtypeStruct((1000, HID), jnp.float32),
            jax.ShapeDtypeStruct((50, HID), jnp.float32),
            jax.ShapeDtypeStruct((20000, HID), jnp.float32),
            jax.ShapeDtypeStruct((500, HID), jnp.float32),
            jax.ShapeDtypeStruct((100, HID), jnp.float32),
        ],
    )(de0, de1, de2, wdev, ee0, ee1, wev, ntt)


_MESH = plsc.VectorSubcoreMesh(
    core_axis_name="c", subcore_axis_name="s", num_cores=2, num_subcores=16)


def _sc_body(d_nid, d_i0, d_i1, d_i2, i_nid, u_nid, n_nid, e_nid, e_i0, e_i1,
             t0p, t1, t2, te0p, te1, ud0, ud1, wu, bu, ntt, out,
             st_nid, st_a, st_b, st_c,
             acc0, acc1, acc2, bb0, bb1, bb2, cc0, cc1, cc2,
             cbuf, udv0, udv1, wuv, buv, nttv,
             gs0, gs1, gs2, ss0, ss1, ss2):
    wid = lax.axis_index("s") * 2 + lax.axis_index("c")
    accs = (acc0, acc1, acc2)
    bbs = (bb0, bb1, bb2)
    ccs = (cc0, cc1, cc2)
    gsems = (gs0, gs1, gs2)
    ssems = (ss0, ss1, ss2)

    def ranges(cpw, s, nc, rot):
        ws = lax.rem(wid + rot, NW)
        c_lo = ws * cpw
        cl = jnp.minimum((c_lo // 8) * 8, nc - s)
        return c_lo, pl.multiple_of(cl, 8)

    def run_pipelined(cpw, nc, c_lo, cl, mk_gathers, compute=None):
        ngrp = _cdiv(cpw, NBUF)

        def pred(s):
            return (s >= 0) & (s < cpw) & ((c_lo + s) < nc)

        def mk_scatter(r, b):
            return pltpu.make_async_copy(accs[b], out.at[st_nid.at[r]],
                                         ssems[b])

        def g_issue(s, b):
            @pl.when(pred(s))
            def _():
                c = c_lo + s
                for d in mk_gathers(c - cl, c, b):
                    d.start()

        def g_wait(s, b):
            @pl.when(pred(s))
            def _():
                c = c_lo + s
                for d in mk_gathers(c - cl, c, b):
                    d.wait()

        def s_issue(s, b):
            @pl.when(pred(s))
            def _():
                mk_scatter(c_lo + s - cl, b).start()

        def s_wait(s, b):
            @pl.when(pred(s))
            def _():
                mk_scatter(c_lo + s - cl, b).wait()

        g_issue(jnp.int32(0), 0)
        g_issue(jnp.int32(1), 1)

        def grp(g, _):
            s0 = g * NBUF
            for bi in range(NBUF):
                s = s0 + bi
                g_wait(s, bi)
                if compute is not None:
                    @pl.when(pred(s))
                    def _(s=s, bi=bi):
                        compute(s, bi)
                s_issue(s, bi)
                s_wait(s - 1, (bi - 1) % NBUF)
                g_issue(s + 2, (bi + 2) % NBUF)
            return 0
        lax.fori_loop(0, ngrp, grp, 0)
        s_wait(ngrp * NBUF - 1, (NBUF - 1) % NBUF)

    def run_const(cpw, nc, c_lo, cl):
        def issue(t, _):
            c = c_lo + t

            @pl.when(c < nc)
            def _():
                pltpu.async_copy(cbuf, out.at[st_nid.at[c - cl]], ss0)
            return 0
        lax.fori_loop(0, cpw, issue, 0)

        def drain(t, _):
            c = c_lo + t

            @pl.when(c < nc)
            def _():
                pltpu.make_async_copy(cbuf, out.at[st_nid.at[c - cl]],
                                      ss0).wait()
            return 0
        lax.fori_loop(0, cpw, drain, 0)

    def fill_cbuf(type_row):
        @plsc.parallel_loop(0, B, unroll=4)
        def _(j):
            for k in range(4):
                sl = pl.ds(k * 16, 16)
                cbuf[j, sl] = nttv[type_row, sl]

    pltpu.sync_copy(ntt, nttv)
    pltpu.sync_copy(wu, wuv)
    pltpu.sync_copy(bu, buv)

    _scope_dev = jax.named_scope("seg_dev"); _scope_dev.__enter__()
    c_lo, cl = ranges(CPW_DEV, S_DEV, C_DEV, 0)
    pltpu.sync_copy(d_nid.at[pl.ds(cl, S_DEV)], st_nid.at[pl.ds(0, S_DEV)])
    pltpu.sync_copy(d_i0.at[pl.ds(cl, S_DEV)], st_a)
    pltpu.sync_copy(d_i1.at[pl.ds(cl, S_DEV)], st_b)
    pltpu.sync_copy(d_i2.at[pl.ds(cl, S_DEV)], st_c)

    def dev_gathers(r, c, b):
        return [pltpu.make_async_copy(t0p.at[st_a.at[r]], accs[b], gsems[b]),
                pltpu.make_async_copy(t1.at[st_b.at[r]], bbs[b], gsems[b]),
                pltpu.make_async_copy(t2.at[st_c.at[r]], ccs[b], gsems[b])]

    def dev_compute(s, b):
        a, bbuf, cbuf2 = accs[b], bbs[b], ccs[b]

        @plsc.parallel_loop(0, B, unroll=4)
        def _(j):
            for k in range(4):
                sl = pl.ds(k * 16, 16)
                plsc.addupdate(a.at[j, sl], bbuf[j, sl] + cbuf2[j, sl])

    run_pipelined(CPW_DEV, C_DEV, c_lo, cl, dev_gathers, dev_compute)

    _scope_dev.__exit__(None, None, None)
    _scope_ip = jax.named_scope("seg_ip"); _scope_ip.__enter__()
    c_lo, cl = ranges(CPW_IP, S_IP, C_IP, 7)
    fill_cbuf(1)
    pltpu.sync_copy(i_nid.at[pl.ds(cl, S_IP)], st_nid.at[pl.ds(0, S_IP)])
    run_const(CPW_IP, C_IP, c_lo, cl)

    _scope_ip.__exit__(None, None, None)
    _scope_us = jax.named_scope("seg_user"); _scope_us.__enter__()
    c_lo, cl = ranges(CPW_USER, S_USER, C_USER, 13)
    pltpu.sync_copy(u_nid.at[pl.ds(cl, S_USER)], st_nid.at[pl.ds(0, S_USER)])
    pltpu.sync_copy(ud0.at[pl.ds(cl * B, S_USER * B)], udv0)
    pltpu.sync_copy(ud1.at[pl.ds(cl * B, S_USER * B)], udv1)

    def user_compute(s, b):
        a = accs[b]
        base = (c_lo + s - cl) * B

        @plsc.parallel_loop(0, B // 16, unroll=2)
        def _(m):
            u0v = udv0[pl.ds(base + m * 16, 16)]
            u1v = udv1[pl.ds(base + m * 16, 16)]
            for jj in range(16):
                j = m * 16 + jj
                for k in range(4):
                    sl = pl.ds(k * 16, 16)
                    a[j, sl] = (u0v[jj] * wuv[0, sl]
                                + u1v[jj] * wuv[1, sl] + buv[0, sl])

    run_pipelined(CPW_USER, C_USER, c_lo, cl, lambda r, c, b: [],
                  user_compute)

    _scope_us.__exit__(None, None, None)
    _scope_nt = jax.named_scope("seg_note"); _scope_nt.__enter__()
    c_lo, cl = ranges(CPW_NOTE, S_NOTE, C_NOTE, 19)
    fill_cbuf(3)
    pltpu.sync_copy(n_nid.at[pl.ds(cl, S_NOTE)], st_nid)
    run_const(CPW_NOTE, C_NOTE, c_lo, cl)

    _scope_nt.__exit__(None, None, None)
    _scope_ev = jax.named_scope("seg_ev"); _scope_ev.__enter__()
    c_lo, cl = ranges(CPW_EV, S_EV, C_EV, 26)
    pltpu.sync_copy(e_nid.at[pl.ds(cl, S_EV)], st_nid.at[pl.ds(0, S_EV)])
    pltpu.sync_copy(e_i0.at[pl.ds(cl, S_EV)], st_a.at[pl.ds(0, S_EV)])
    pltpu.sync_copy(e_i1.at[pl.ds(cl, S_EV)], st_b.at[pl.ds(0, S_EV)])

    def ev_gathers(r, c, b):
        return [pltpu.make_async_copy(te0p.at[st_a.at[r]], accs[b], gsems[b]),
                pltpu.make_async_copy(te1.at[st_b.at[r]], bbs[b], gsems[b])]

    def ev_compute(s, b):
        a, bbuf = accs[b], bbs[b]

        @plsc.parallel_loop(0, B, unroll=4)
        def _(j):
            for k in range(4):
                sl = pl.ds(k * 16, 16)
                plsc.addupdate(a.at[j, sl], bbuf[j, sl])

    run_pipelined(CPW_EV, C_EV, c_lo, cl, ev_gathers, ev_compute)
    _scope_ev.__exit__(None, None, None)


N_PAD = 3907 * B

_sc_scatter = functools.partial(
    pl.kernel,
    out_type=jax.ShapeDtypeStruct((N_PAD, HID), jnp.float32),
    mesh=_MESH,
    scratch_types=[
        pltpu.VMEM((S_NOTE, B), jnp.int32),
        pltpu.VMEM((S_DEV, B), jnp.int32),
        pltpu.VMEM((S_DEV, B), jnp.int32),
        pltpu.VMEM((S_DEV, B), jnp.int32),
        pltpu.VMEM((B, HID), jnp.float32),
        pltpu.VMEM((B, HID), jnp.float32),
        pltpu.VMEM((B, HID), jnp.float32),
        pltpu.VMEM((B, HID), jnp.float32),
        pltpu.VMEM((B, HID), jnp.float32),
        pltpu.VMEM((B, HID), jnp.float32),
        pltpu.VMEM((B, HID), jnp.float32),
        pltpu.VMEM((B, HID), jnp.float32),
        pltpu.VMEM((B, HID), jnp.float32),
        pltpu.VMEM((B, HID), jnp.float32),
        pltpu.VMEM((S_USER * B,), jnp.float32),
        pltpu.VMEM((S_USER * B,), jnp.float32),
        pltpu.VMEM((2, HID), jnp.float32),
        pltpu.VMEM((1, HID), jnp.float32),
        pltpu.VMEM((5, HID), jnp.float32),
        pltpu.SemaphoreType.DMA,
        pltpu.SemaphoreType.DMA,
        pltpu.SemaphoreType.DMA,
        pltpu.SemaphoreType.DMA,
        pltpu.SemaphoreType.DMA,
        pltpu.SemaphoreType.DMA,
    ],
    compiler_params=pltpu.CompilerParams(use_tc_tiling_on_sc=False),
)(_sc_body)


NCH = N_PAD // B
CPW_T = _cdiv(NCH, NW)


def _tr_body(src, t4, sb0, sb1, sb2, db0, db1, db2,
             rs0, rs1, rs2, ws0, ws1, ws2):
    wid = lax.axis_index("s") * 2 + lax.axis_index("c")
    sbs = (sb0, sb1, sb2)
    dbs = (db0, db1, db2)
    rsems = (rs0, rs1, rs2)
    wsems = (ws0, ws1, ws2)
    c_lo = wid * CPW_T
    base_iota = lax.iota(jnp.int32, 16)

    def pred(s):
        return (s >= 0) & (s < CPW_T) & ((c_lo + s) < NCH)

    ivm = [base_iota * HID + m * (16 * HID) for m in range(8)]

    def mk_read(c, b):
        return pltpu.make_async_copy(src.at[pl.ds(c * B * HID, B * HID)],
                                     sbs[b], rsems[b])

    def mk_write(c, b):
        return pltpu.make_async_copy(dbs[b], t4.at[:, c], wsems[b])

    def r_issue(s, b):
        @pl.when(pred(s))
        def _():
            mk_read(c_lo + s, b).start()

    def r_wait(s, b):
        @pl.when(pred(s))
        def _():
            mk_read(c_lo + s, b).wait()

    def w_issue(s, b):
        @pl.when(pred(s))
        def _():
            mk_write(c_lo + s, b).start()

    def w_wait(s, b):
        @pl.when(pred(s))
        def _():
            mk_write(c_lo + s, b).wait()

    def transpose(b):
        sb, db = sbs[b], dbs[b]

        @plsc.parallel_loop(0, 8, unroll=1)
        def _(i):
            for s8 in range(8):
                off = i * 8 + s8
                for m in range(8):
                    g = plsc.load_gather(sb, [ivm[m] + off])
                    db[i, s8, pl.ds(m * 16, 16)] = g

    r_issue(jnp.int32(0), 0)
    r_issue(jnp.int32(1), 1)

    def grp(g, _):
        s0 = g * NBUF
        for bi in range(NBUF):
            s = s0 + bi
            r_wait(s, bi)

            @pl.when(pred(s))
            def _(bi=bi):
                transpose(bi)
            w_issue(s, bi)
            w_wait(s - 1, (bi - 1) % NBUF)
            r_issue(s + 2, (bi + 2) % NBUF)
        return 0
    lax.fori_loop(0, _cdiv(CPW_T, NBUF), grp, 0)
    w_wait(_cdiv(CPW_T, NBUF) * NBUF - 1, (NBUF - 1) % NBUF)


_sc_transpose = functools.partial(
    pl.kernel,
    out_type=jax.ShapeDtypeStruct((8, NCH, 8, B), jnp.float32),
    mesh=_MESH,
    scratch_types=[
        pltpu.VMEM((B * HID,), jnp.float32),
        pltpu.VMEM((B * HID,), jnp.float32),
        pltpu.VMEM((B * HID,), jnp.float32),
        pltpu.VMEM((8, 8, B), jnp.float32),
        pltpu.VMEM((8, 8, B), jnp.float32),
        pltpu.VMEM((8, 8, B), jnp.float32),
        pltpu.SemaphoreType.DMA,
        pltpu.SemaphoreType.DMA,
        pltpu.SemaphoreType.DMA,
        pltpu.SemaphoreType.DMA,
        pltpu.SemaphoreType.DMA,
        pltpu.SemaphoreType.DMA,
    ],
    compiler_params=pltpu.CompilerParams(use_tc_tiling_on_sc=False,
                                         needs_layout_passes=False),
)(_tr_body)


def _pad_chunks(x, nc):
    return jnp.pad(x, (0, nc * B - x.shape[0]), mode='edge').reshape(nc, B)


def kernel(device_node_id, ip_node_id, user_node_id, note_node_id,
           event_node_id, device_cat, event_cat, user_dense,
           node_type_table, dev_emb0, dev_emb1, dev_emb2, W_dev,
           ev_emb0, ev_emb1, W_ev, W_user, b_user):
    i32 = jnp.int32
    d_nid = _pad_chunks(device_node_id.astype(i32), C_DEV)
    d_i0 = _pad_chunks(device_cat[:, 0].astype(i32), C_DEV)
    d_i1 = _pad_chunks(device_cat[:, 1].astype(i32), C_DEV)
    d_i2 = _pad_chunks(device_cat[:, 2].astype(i32), C_DEV)
    i_nid = _pad_chunks(ip_node_id.astype(i32), C_IP)
    u_nid = _pad_chunks(user_node_id.astype(i32), C_USER)
    n_nid = _pad_chunks(note_node_id.astype(i32), C_NOTE)
    e_nid = _pad_chunks(event_node_id.astype(i32), C_EV)
    e_i0 = _pad_chunks(event_cat[:, 0].astype(i32), C_EV)
    e_i1 = _pad_chunks(event_cat[:, 1].astype(i32), C_EV)

    t0p, t1, t2, te0p, te1 = _tc_tables(
        dev_emb0, dev_emb1, dev_emb2, W_dev, ev_emb0, ev_emb1, W_ev,
        node_type_table)

    npad = C_USER * B - N_USER
    ud0 = jnp.pad(user_dense[:, 0], (0, npad), mode='edge')
    ud1 = jnp.pad(user_dense[:, 1], (0, npad), mode='edge')

    dense = _sc_scatter(d_nid, d_i0, d_i1, d_i2, i_nid, u_nid, n_nid,
                        e_nid, e_i0, e_i1, t0p, t1, t2, te0p, te1,
                        ud0, ud1, W_user, b_user.reshape(1, HID),
                        node_type_table)
    t4 = _sc_transpose(dense.reshape(N_PAD * HID))
    return t4.transpose(1, 3, 0, 2).reshape(N_PAD, HID)[:N_TOTAL]


# speedup vs baseline: 1.2698x; 1.2160x over previous
import functools

import jax
import jax.numpy as jnp
from jax import lax
from jax.experimental import pallas as pl
from jax.experimental.pallas import tpu as pltpu
from jax.experimental.pallas import tpu_sc as plsc

HID = 64
B = 128
NW = 32
NBUF = 3

N_DEV = 100000
N_IP = 100000
N_USER = 100000
N_NOTE = 150000
N_EV = 50000
N_TOTAL = 500000


def _cdiv(a, b):
    return (a + b - 1) // b


def _pad8(n):
    return _cdiv(n, 8) * 8


C_DEV = _pad8(_cdiv(N_DEV, B))
C_IP = _pad8(_cdiv(N_IP, B))
C_USER = _pad8(_cdiv(N_USER, B))
C_NOTE = _pad8(_cdiv(N_NOTE, B))
C_EV = _pad8(_cdiv(N_EV, B))

CPW_DEV = _cdiv(C_DEV, NW)
CPW_IP = _cdiv(C_IP, NW)
CPW_USER = _cdiv(C_USER, NW)
CPW_NOTE = _cdiv(C_NOTE, NW)
CPW_EV = _cdiv(C_EV, NW)


def _stage_rows(cpw):
    return (_cdiv(cpw, 8) + 1) * 8


S_DEV = _stage_rows(CPW_DEV)
S_IP = _stage_rows(CPW_IP)
S_USER = _stage_rows(CPW_USER)
S_NOTE = _stage_rows(CPW_NOTE)
S_EV = _stage_rows(CPW_EV)


def _tables_body(de0, de1, de2, wdev, ee0, ee1, wev, ntt,
                 t0p, t1, t2, te0p, te1):
    wd = wdev[...]
    we = wev[...]
    n = ntt[...]
    t0p[...] = jnp.dot(de0[...], wd[0:16, :],
                       preferred_element_type=jnp.float32) + n[0:1, :]
    t1[...] = jnp.dot(de1[...], wd[16:24, :],
                      preferred_element_type=jnp.float32)
    t2[...] = jnp.dot(de2[...], wd[24:40, :],
                      preferred_element_type=jnp.float32)
    te0p[...] = jnp.dot(ee0[...], we[0:16, :],
                        preferred_element_type=jnp.float32) + n[4:5, :]
    te1[...] = jnp.dot(ee1[...], we[16:24, :],
                       preferred_element_type=jnp.float32)


def _tc_tables(de0, de1, de2, wdev, ee0, ee1, wev, ntt):
    return pl.pallas_call(
        _tables_body,
        out_shape=[
            jax.ShapeDtypeStruct((1000, HID), jnp.float32),
            jax.ShapeDtypeStruct((50, HID), jnp.float32),
            jax.ShapeDtypeStruct((20000, HID), jnp.float32),
            jax.ShapeDtypeStruct((500, HID), jnp.float32),
            jax.ShapeDtypeStruct((100, HID), jnp.float32),
        ],
    )(de0, de1, de2, wdev, ee0, ee1, wev, ntt)


_MESH = plsc.VectorSubcoreMesh(
    core_axis_name="c", subcore_axis_name="s", num_cores=2, num_subcores=16)


def _sc_body(d_nid, d_i0, d_i1, d_i2, i_nid, u_nid, n_nid, e_nid, e_i0, e_i1,
             t0p, t1, t2, te0p, te1, ud0, ud1, wu, bu, ntt, out,
             st_nid, st_a, st_b, st_c,
             acc0, acc1, acc2, bb0, bb1, bb2, cc0, cc1, cc2,
             cbuf, udv0, udv1, wuv, buv, nttv,
             gs0, gs1, gs2, ss0, ss1, ss2):
    wid = lax.axis_index("s") * 2 + lax.axis_index("c")
    accs = (acc0, acc1, acc2)
    bbs = (bb0, bb1, bb2)
    ccs = (cc0, cc1, cc2)
    gsems = (gs0, gs1, gs2)
    ssems = (ss0, ss1, ss2)

    def ranges(cpw, s, nc, rot):
        ws = lax.rem(wid + rot, NW)
        c_lo = ws * cpw
        cl = jnp.minimum((c_lo // 8) * 8, nc - s)
        return c_lo, pl.multiple_of(cl, 8)

    def run_pipelined(cpw, nc, c_lo, cl, mk_gathers, compute=None):
        ngrp = _cdiv(cpw, NBUF)

        def pred(s):
            return (s >= 0) & (s < cpw) & ((c_lo + s) < nc)

        def mk_scatter(r, b):
            return pltpu.make_async_copy(accs[b], out.at[st_nid.at[r]],
                                         ssems[b])

        def g_issue(s, b):
            @pl.when(pred(s))
            def _():
                c = c_lo + s
                for d in mk_gathers(c - cl, c, b):
                    d.start()

        def g_wait(s, b):
            @pl.when(pred(s))
            def _():
                c = c_lo + s
                for d in mk_gathers(c - cl, c, b):
                    d.wait()

        def s_issue(s, b):
            @pl.when(pred(s))
            def _():
                mk_scatter(c_lo + s - cl, b).start()

        def s_wait(s, b):
            @pl.when(pred(s))
            def _():
                mk_scatter(c_lo + s - cl, b).wait()

        g_issue(jnp.int32(0), 0)
        g_issue(jnp.int32(1), 1)

        def grp(g, _):
            s0 = g * NBUF
            for bi in range(NBUF):
                s = s0 + bi
                g_wait(s, bi)
                if compute is not None:
                    @pl.when(pred(s))
                    def _(s=s, bi=bi):
                        compute(s, bi)
                s_issue(s, bi)
                s_wait(s - 1, (bi - 1) % NBUF)
                g_issue(s + 2, (bi + 2) % NBUF)
            return 0
        lax.fori_loop(0, ngrp, grp, 0)
        s_wait(ngrp * NBUF - 1, (NBUF - 1) % NBUF)

    def run_const(cpw, nc, c_lo, cl):
        def issue(t, _):
            c = c_lo + t

            @pl.when(c < nc)
            def _():
                pltpu.async_copy(cbuf, out.at[st_nid.at[c - cl]], ss0)
            return 0
        lax.fori_loop(0, cpw, issue, 0)

        def drain(t, _):
            c = c_lo + t

            @pl.when(c < nc)
            def _():
                pltpu.make_async_copy(cbuf, out.at[st_nid.at[c - cl]],
                                      ss0).wait()
            return 0
        lax.fori_loop(0, cpw, drain, 0)

    def fill_cbuf(type_row):
        @plsc.parallel_loop(0, B, unroll=4)
        def _(j):
            for k in range(4):
                sl = pl.ds(k * 16, 16)
                cbuf[j, sl] = nttv[type_row, sl]

    pltpu.sync_copy(ntt, nttv)
    pltpu.sync_copy(wu, wuv)
    pltpu.sync_copy(bu, buv)

    _scope_dev = jax.named_scope("seg_dev"); _scope_dev.__enter__()
    c_lo, cl = ranges(CPW_DEV, S_DEV, C_DEV, 0)
    pltpu.sync_copy(d_nid.at[pl.ds(cl, S_DEV)], st_nid.at[pl.ds(0, S_DEV)])
    pltpu.sync_copy(d_i0.at[pl.ds(cl, S_DEV)], st_a)
    pltpu.sync_copy(d_i1.at[pl.ds(cl, S_DEV)], st_b)
    pltpu.sync_copy(d_i2.at[pl.ds(cl, S_DEV)], st_c)

    def dev_gathers(r, c, b):
        return [pltpu.make_async_copy(t0p.at[st_a.at[r]], accs[b], gsems[b]),
                pltpu.make_async_copy(t1.at[st_b.at[r]], bbs[b], gsems[b]),
                pltpu.make_async_copy(t2.at[st_c.at[r]], ccs[b], gsems[b])]

    def dev_compute(s, b):
        a, bbuf, cbuf2 = accs[b], bbs[b], ccs[b]

        @plsc.parallel_loop(0, B, unroll=4)
        def _(j):
            for k in range(4):
                sl = pl.ds(k * 16, 16)
                plsc.addupdate(a.at[j, sl], bbuf[j, sl] + cbuf2[j, sl])

    run_pipelined(CPW_DEV, C_DEV, c_lo, cl, dev_gathers, dev_compute)

    _scope_dev.__exit__(None, None, None)
    _scope_ip = jax.named_scope("seg_ip"); _scope_ip.__enter__()
    c_lo, cl = ranges(CPW_IP, S_IP, C_IP, 7)
    fill_cbuf(1)
    pltpu.sync_copy(i_nid.at[pl.ds(cl, S_IP)], st_nid.at[pl.ds(0, S_IP)])
    run_const(CPW_IP, C_IP, c_lo, cl)

    _scope_ip.__exit__(None, None, None)
    _scope_us = jax.named_scope("seg_user"); _scope_us.__enter__()
    c_lo, cl = ranges(CPW_USER, S_USER, C_USER, 13)
    pltpu.sync_copy(u_nid.at[pl.ds(cl, S_USER)], st_nid.at[pl.ds(0, S_USER)])
    pltpu.sync_copy(ud0.at[pl.ds(cl * B, S_USER * B)], udv0)
    pltpu.sync_copy(ud1.at[pl.ds(cl * B, S_USER * B)], udv1)

    def user_compute(s, b):
        a = accs[b]
        base = (c_lo + s - cl) * B

        @plsc.parallel_loop(0, B // 16, unroll=2)
        def _(m):
            u0v = udv0[pl.ds(base + m * 16, 16)]
            u1v = udv1[pl.ds(base + m * 16, 16)]
            for jj in range(16):
                j = m * 16 + jj
                for k in range(4):
                    sl = pl.ds(k * 16, 16)
                    a[j, sl] = (u0v[jj] * wuv[0, sl]
                                + u1v[jj] * wuv[1, sl] + buv[0, sl])

    run_pipelined(CPW_USER, C_USER, c_lo, cl, lambda r, c, b: [],
                  user_compute)

    _scope_us.__exit__(None, None, None)
    _scope_nt = jax.named_scope("seg_note"); _scope_nt.__enter__()
    c_lo, cl = ranges(CPW_NOTE, S_NOTE, C_NOTE, 19)
    fill_cbuf(3)
    pltpu.sync_copy(n_nid.at[pl.ds(cl, S_NOTE)], st_nid)
    run_const(CPW_NOTE, C_NOTE, c_lo, cl)

    _scope_nt.__exit__(None, None, None)
    _scope_ev = jax.named_scope("seg_ev"); _scope_ev.__enter__()
    c_lo, cl = ranges(CPW_EV, S_EV, C_EV, 26)
    pltpu.sync_copy(e_nid.at[pl.ds(cl, S_EV)], st_nid.at[pl.ds(0, S_EV)])
    pltpu.sync_copy(e_i0.at[pl.ds(cl, S_EV)], st_a.at[pl.ds(0, S_EV)])
    pltpu.sync_copy(e_i1.at[pl.ds(cl, S_EV)], st_b.at[pl.ds(0, S_EV)])

    def ev_gathers(r, c, b):
        return [pltpu.make_async_copy(te0p.at[st_a.at[r]], accs[b], gsems[b]),
                pltpu.make_async_copy(te1.at[st_b.at[r]], bbs[b], gsems[b])]

    def ev_compute(s, b):
        a, bbuf = accs[b], bbs[b]

        @plsc.parallel_loop(0, B, unroll=4)
        def _(j):
            for k in range(4):
                sl = pl.ds(k * 16, 16)
                plsc.addupdate(a.at[j, sl], bbuf[j, sl])

    run_pipelined(CPW_EV, C_EV, c_lo, cl, ev_gathers, ev_compute)
    _scope_ev.__exit__(None, None, None)


N_PAD = 3907 * B

_sc_scatter = functools.partial(
    pl.kernel,
    out_type=jax.ShapeDtypeStruct((N_PAD, HID), jnp.float32),
    mesh=_MESH,
    scratch_types=[
        pltpu.VMEM((S_NOTE, B), jnp.int32),
        pltpu.VMEM((S_DEV, B), jnp.int32),
        pltpu.VMEM((S_DEV, B), jnp.int32),
        pltpu.VMEM((S_DEV, B), jnp.int32),
        pltpu.VMEM((B, HID), jnp.float32),
        pltpu.VMEM((B, HID), jnp.float32),
        pltpu.VMEM((B, HID), jnp.float32),
        pltpu.VMEM((B, HID), jnp.float32),
        pltpu.VMEM((B, HID), jnp.float32),
        pltpu.VMEM((B, HID), jnp.float32),
        pltpu.VMEM((B, HID), jnp.float32),
        pltpu.VMEM((B, HID), jnp.float32),
        pltpu.VMEM((B, HID), jnp.float32),
        pltpu.VMEM((B, HID), jnp.float32),
        pltpu.VMEM((S_USER * B,), jnp.float32),
        pltpu.VMEM((S_USER * B,), jnp.float32),
        pltpu.VMEM((2, HID), jnp.float32),
        pltpu.VMEM((1, HID), jnp.float32),
        pltpu.VMEM((5, HID), jnp.float32),
        pltpu.SemaphoreType.DMA,
        pltpu.SemaphoreType.DMA,
        pltpu.SemaphoreType.DMA,
        pltpu.SemaphoreType.DMA,
        pltpu.SemaphoreType.DMA,
        pltpu.SemaphoreType.DMA,
    ],
    compiler_params=pltpu.CompilerParams(use_tc_tiling_on_sc=False),
)(_sc_body)


NCH = N_PAD // B
CPW_T = _cdiv(NCH, NW)


def _tr_body(src, t4, sb0, sb1, sb2, db0, db1, db2,
             rs0, rs1, rs2, ws0, ws1, ws2):
    wid = lax.axis_index("s") * 2 + lax.axis_index("c")
    sbs = (sb0, sb1, sb2)
    dbs = (db0, db1, db2)
    rsems = (rs0, rs1, rs2)
    wsems = (ws0, ws1, ws2)
    c_lo = wid * CPW_T
    base_iota = lax.iota(jnp.int32, 16)

    def pred(s):
        return (s >= 0) & (s < CPW_T) & ((c_lo + s) < NCH)

    rowvs = [base_iota + m * 16 for m in range(8)]

    def mk_read(c, b):
        return pltpu.make_async_copy(src.at[pl.ds(c * B, B)],
                                     sbs[b].at[:, pl.ds(0, HID)], rsems[b])

    def mk_write(c, b):
        return pltpu.make_async_copy(dbs[b], t4.at[:, c], wsems[b])

    def r_issue(s, b):
        @pl.when(pred(s))
        def _():
            mk_read(c_lo + s, b).start()

    def r_wait(s, b):
        @pl.when(pred(s))
        def _():
            mk_read(c_lo + s, b).wait()

    def w_issue(s, b):
        @pl.when(pred(s))
        def _():
            mk_write(c_lo + s, b).start()

    def w_wait(s, b):
        @pl.when(pred(s))
        def _():
            mk_write(c_lo + s, b).wait()

    def transpose(b):
        sb, db = sbs[b], dbs[b]

        @plsc.parallel_loop(0, 8, unroll=1)
        def _(i):
            for s8 in range(8):
                colv = base_iota * 0 + (i * 8 + s8)
                for m in range(8):
                    g = plsc.load_gather(sb, [rowvs[m], colv])
                    db[i, s8, pl.ds(m * 16, 16)] = g

    r_issue(jnp.int32(0), 0)
    r_issue(jnp.int32(1), 1)

    def grp(g, _):
        s0 = g * NBUF
        for bi in range(NBUF):
            s = s0 + bi
            r_wait(s, bi)

            @pl.when(pred(s))
            def _(bi=bi):
                transpose(bi)
            w_issue(s, bi)
            w_wait(s - 1, (bi - 1) % NBUF)
            r_issue(s + 2, (bi + 2) % NBUF)
        return 0
    lax.fori_loop(0, _cdiv(CPW_T, NBUF), grp, 0)
    w_wait(_cdiv(CPW_T, NBUF) * NBUF - 1, (NBUF - 1) % NBUF)


_sc_transpose = functools.partial(
    pl.kernel,
    out_type=jax.ShapeDtypeStruct((8, NCH, 8, B), jnp.float32),
    mesh=_MESH,
    scratch_types=[
        pltpu.VMEM((B, HID + 1), jnp.float32),
        pltpu.VMEM((B, HID + 1), jnp.float32),
        pltpu.VMEM((B, HID + 1), jnp.float32),
        pltpu.VMEM((8, 8, B), jnp.float32),
        pltpu.VMEM((8, 8, B), jnp.float32),
        pltpu.VMEM((8, 8, B), jnp.float32),
        pltpu.SemaphoreType.DMA,
        pltpu.SemaphoreType.DMA,
        pltpu.SemaphoreType.DMA,
        pltpu.SemaphoreType.DMA,
        pltpu.SemaphoreType.DMA,
        pltpu.SemaphoreType.DMA,
    ],
    compiler_params=pltpu.CompilerParams(use_tc_tiling_on_sc=False,
                                         needs_layout_passes=False),
)(_tr_body)


def _pad_chunks(x, nc):
    return jnp.pad(x, (0, nc * B - x.shape[0]), mode='edge').reshape(nc, B)


def kernel(device_node_id, ip_node_id, user_node_id, note_node_id,
           event_node_id, device_cat, event_cat, user_dense,
           node_type_table, dev_emb0, dev_emb1, dev_emb2, W_dev,
           ev_emb0, ev_emb1, W_ev, W_user, b_user):
    i32 = jnp.int32
    d_nid = _pad_chunks(device_node_id.astype(i32), C_DEV)
    d_i0 = _pad_chunks(device_cat[:, 0].astype(i32), C_DEV)
    d_i1 = _pad_chunks(device_cat[:, 1].astype(i32), C_DEV)
    d_i2 = _pad_chunks(device_cat[:, 2].astype(i32), C_DEV)
    i_nid = _pad_chunks(ip_node_id.astype(i32), C_IP)
    u_nid = _pad_chunks(user_node_id.astype(i32), C_USER)
    n_nid = _pad_chunks(note_node_id.astype(i32), C_NOTE)
    e_nid = _pad_chunks(event_node_id.astype(i32), C_EV)
    e_i0 = _pad_chunks(event_cat[:, 0].astype(i32), C_EV)
    e_i1 = _pad_chunks(event_cat[:, 1].astype(i32), C_EV)

    t0p, t1, t2, te0p, te1 = _tc_tables(
        dev_emb0, dev_emb1, dev_emb2, W_dev, ev_emb0, ev_emb1, W_ev,
        node_type_table)

    npad = C_USER * B - N_USER
    ud0 = jnp.pad(user_dense[:, 0], (0, npad), mode='edge')
    ud1 = jnp.pad(user_dense[:, 1], (0, npad), mode='edge')

    dense = _sc_scatter(d_nid, d_i0, d_i1, d_i2, i_nid, u_nid, n_nid,
                        e_nid, e_i0, e_i1, t0p, t1, t2, te0p, te1,
                        ud0, ud1, W_user, b_user.reshape(1, HID),
                        node_type_table)
    t4 = _sc_transpose(dense)
    return t4.transpose(1, 3, 0, 2).reshape(N_PAD, HID)[:N_TOTAL]


# speedup vs baseline: 1.3010x; 1.0245x over previous
import functools

import jax
import jax.numpy as jnp
from jax import lax
from jax.experimental import pallas as pl
from jax.experimental.pallas import tpu as pltpu
from jax.experimental.pallas import tpu_sc as plsc

HID = 64
B = 128
NW = 32
NBUF = 3

N_DEV = 100000
N_IP = 100000
N_USER = 100000
N_NOTE = 150000
N_EV = 50000
N_TOTAL = 500000


def _cdiv(a, b):
    return (a + b - 1) // b


def _pad8(n):
    return _cdiv(n, 8) * 8


C_DEV = _pad8(_cdiv(N_DEV, B))
C_IP = _pad8(_cdiv(N_IP, B))
C_USER = _pad8(_cdiv(N_USER, B))
C_NOTE = _pad8(_cdiv(N_NOTE, B))
C_EV = _pad8(_cdiv(N_EV, B))

CPW_DEV = _cdiv(C_DEV, NW)
CPW_IP = _cdiv(C_IP, NW)
CPW_USER = _cdiv(C_USER, NW)
CPW_NOTE = _cdiv(C_NOTE, NW)
CPW_EV = _cdiv(C_EV, NW)


def _stage_rows(cpw):
    return (_cdiv(cpw, 8) + 1) * 8


S_DEV = _stage_rows(CPW_DEV)
S_IP = _stage_rows(CPW_IP)
S_USER = _stage_rows(CPW_USER)
S_NOTE = _stage_rows(CPW_NOTE)
S_EV = _stage_rows(CPW_EV)


def _tables_body(de0, de1, de2, wdev, ee0, ee1, wev, ntt,
                 t0p, t1, t2, te0p, te1):
    wd = wdev[...]
    we = wev[...]
    n = ntt[...]
    t0p[...] = jnp.dot(de0[...], wd[0:16, :],
                       preferred_element_type=jnp.float32) + n[0:1, :]
    t1[...] = jnp.dot(de1[...], wd[16:24, :],
                      preferred_element_type=jnp.float32)
    t2[...] = jnp.dot(de2[...], wd[24:40, :],
                      preferred_element_type=jnp.float32)
    te0p[...] = jnp.dot(ee0[...], we[0:16, :],
                        preferred_element_type=jnp.float32) + n[4:5, :]
    te1[...] = jnp.dot(ee1[...], we[16:24, :],
                       preferred_element_type=jnp.float32)


def _tc_tables(de0, de1, de2, wdev, ee0, ee1, wev, ntt):
    return pl.pallas_call(
        _tables_body,
        out_shape=[
            jax.ShapeDtypeStruct((1000, HID), jnp.float32),
            jax.ShapeDtypeStruct((50, HID), jnp.float32),
            jax.ShapeDtypeStruct((20000, HID), jnp.float32),
            jax.ShapeDtypeStruct((500, HID), jnp.float32),
            jax.ShapeDtypeStruct((100, HID), jnp.float32),
        ],
    )(de0, de1, de2, wdev, ee0, ee1, wev, ntt)


_MESH = plsc.VectorSubcoreMesh(
    core_axis_name="c", subcore_axis_name="s", num_cores=2, num_subcores=16)


def _sc_body(d_nid, d_i0, d_i1, d_i2, i_nid, u_nid, n_nid, e_nid, e_i0, e_i1,
             t0p, t1, t2, te0p, te1, ud0, ud1, wu, bu, ntt, out,
             st_nid, st_a, st_b, st_c,
             acc0, acc1, acc2, bb0, bb1, bb2, cc0, cc1, cc2,
             cbuf, udv0, udv1, wuv, buv, nttv,
             gs0, gs1, gs2, ss0, ss1, ss2):
    wid = lax.axis_index("s") * 2 + lax.axis_index("c")
    accs = (acc0, acc1, acc2)
    bbs = (bb0, bb1, bb2)
    ccs = (cc0, cc1, cc2)
    gsems = (gs0, gs1, gs2)
    ssems = (ss0, ss1, ss2)

    def ranges(cpw, s, nc, rot):
        ws = lax.rem(wid + rot, NW)
        c_lo = ws * cpw
        cl = jnp.minimum((c_lo // 8) * 8, nc - s)
        return c_lo, pl.multiple_of(cl, 8)

    def run_pipelined(cpw, nc, c_lo, cl, mk_gathers, compute=None):
        ngrp = _cdiv(cpw, NBUF)

        def pred(s):
            return (s >= 0) & (s < cpw) & ((c_lo + s) < nc)

        def mk_scatter(r, b):
            return pltpu.make_async_copy(accs[b], out.at[st_nid.at[r]],
                                         ssems[b])

        def g_issue(s, b):
            @pl.when(pred(s))
            def _():
                c = c_lo + s
                for d in mk_gathers(c - cl, c, b):
                    d.start()

        def g_wait(s, b):
            @pl.when(pred(s))
            def _():
                c = c_lo + s
                for d in mk_gathers(c - cl, c, b):
                    d.wait()

        def s_issue(s, b):
            @pl.when(pred(s))
            def _():
                mk_scatter(c_lo + s - cl, b).start()

        def s_wait(s, b):
            @pl.when(pred(s))
            def _():
                mk_scatter(c_lo + s - cl, b).wait()

        g_issue(jnp.int32(0), 0)
        g_issue(jnp.int32(1), 1)

        def grp(g, _):
            s0 = g * NBUF
            for bi in range(NBUF):
                s = s0 + bi
                g_wait(s, bi)
                if compute is not None:
                    @pl.when(pred(s))
                    def _(s=s, bi=bi):
                        compute(s, bi)
                s_issue(s, bi)
                s_wait(s - 1, (bi - 1) % NBUF)
                g_issue(s + 2, (bi + 2) % NBUF)
            return 0
        lax.fori_loop(0, ngrp, grp, 0)
        s_wait(ngrp * NBUF - 1, (NBUF - 1) % NBUF)

    def run_const(cpw, nc, c_lo, cl):
        def issue(t, _):
            c = c_lo + t

            @pl.when(c < nc)
            def _():
                pltpu.async_copy(cbuf, out.at[st_nid.at[c - cl]], ss0)
            return 0
        lax.fori_loop(0, cpw, issue, 0)

        def drain(t, _):
            c = c_lo + t

            @pl.when(c < nc)
            def _():
                pltpu.make_async_copy(cbuf, out.at[st_nid.at[c - cl]],
                                      ss0).wait()
            return 0
        lax.fori_loop(0, cpw, drain, 0)

    def fill_cbuf(type_row):
        @plsc.parallel_loop(0, B, unroll=4)
        def _(j):
            for k in range(4):
                sl = pl.ds(k * 16, 16)
                cbuf[j, sl] = nttv[type_row, sl]

    pltpu.sync_copy(ntt, nttv)
    pltpu.sync_copy(wu, wuv)
    pltpu.sync_copy(bu, buv)

    _scope_dev = jax.named_scope("seg_dev"); _scope_dev.__enter__()
    c_lo, cl = ranges(CPW_DEV, S_DEV, C_DEV, 0)
    pltpu.sync_copy(d_nid.at[pl.ds(cl, S_DEV)], st_nid.at[pl.ds(0, S_DEV)])
    pltpu.sync_copy(d_i0.at[pl.ds(cl, S_DEV)], st_a)
    pltpu.sync_copy(d_i1.at[pl.ds(cl, S_DEV)], st_b)
    pltpu.sync_copy(d_i2.at[pl.ds(cl, S_DEV)], st_c)

    def dev_gathers(r, c, b):
        return [pltpu.make_async_copy(t0p.at[st_a.at[r]], accs[b], gsems[b]),
                pltpu.make_async_copy(t1.at[st_b.at[r]], bbs[b], gsems[b]),
                pltpu.make_async_copy(t2.at[st_c.at[r]], ccs[b], gsems[b])]

    def dev_compute(s, b):
        a, bbuf, cbuf2 = accs[b], bbs[b], ccs[b]

        @plsc.parallel_loop(0, B, unroll=4)
        def _(j):
            for k in range(4):
                sl = pl.ds(k * 16, 16)
                plsc.addupdate(a.at[j, sl], bbuf[j, sl] + cbuf2[j, sl])

    run_pipelined(CPW_DEV, C_DEV, c_lo, cl, dev_gathers, dev_compute)

    _scope_dev.__exit__(None, None, None)
    _scope_ip = jax.named_scope("seg_ip"); _scope_ip.__enter__()
    c_lo, cl = ranges(CPW_IP, S_IP, C_IP, 7)
    fill_cbuf(1)
    pltpu.sync_copy(i_nid.at[pl.ds(cl, S_IP)], st_nid.at[pl.ds(0, S_IP)])
    run_const(CPW_IP, C_IP, c_lo, cl)

    _scope_ip.__exit__(None, None, None)
    _scope_us = jax.named_scope("seg_user"); _scope_us.__enter__()
    c_lo, cl = ranges(CPW_USER, S_USER, C_USER, 13)
    pltpu.sync_copy(u_nid.at[pl.ds(cl, S_USER)], st_nid.at[pl.ds(0, S_USER)])
    pltpu.sync_copy(ud0.at[pl.ds(cl * B, S_USER * B)], udv0)
    pltpu.sync_copy(ud1.at[pl.ds(cl * B, S_USER * B)], udv1)

    def user_compute(s, b):
        a = accs[b]
        base = (c_lo + s - cl) * B

        @plsc.parallel_loop(0, B // 16, unroll=2)
        def _(m):
            u0v = udv0[pl.ds(base + m * 16, 16)]
            u1v = udv1[pl.ds(base + m * 16, 16)]
            for jj in range(16):
                j = m * 16 + jj
                for k in range(4):
                    sl = pl.ds(k * 16, 16)
                    a[j, sl] = (u0v[jj] * wuv[0, sl]
                                + u1v[jj] * wuv[1, sl] + buv[0, sl])

    run_pipelined(CPW_USER, C_USER, c_lo, cl, lambda r, c, b: [],
                  user_compute)

    _scope_us.__exit__(None, None, None)
    _scope_nt = jax.named_scope("seg_note"); _scope_nt.__enter__()
    c_lo, cl = ranges(CPW_NOTE, S_NOTE, C_NOTE, 19)
    fill_cbuf(3)
    pltpu.sync_copy(n_nid.at[pl.ds(cl, S_NOTE)], st_nid)
    run_const(CPW_NOTE, C_NOTE, c_lo, cl)

    _scope_nt.__exit__(None, None, None)
    _scope_ev = jax.named_scope("seg_ev"); _scope_ev.__enter__()
    c_lo, cl = ranges(CPW_EV, S_EV, C_EV, 26)
    pltpu.sync_copy(e_nid.at[pl.ds(cl, S_EV)], st_nid.at[pl.ds(0, S_EV)])
    pltpu.sync_copy(e_i0.at[pl.ds(cl, S_EV)], st_a.at[pl.ds(0, S_EV)])
    pltpu.sync_copy(e_i1.at[pl.ds(cl, S_EV)], st_b.at[pl.ds(0, S_EV)])

    def ev_gathers(r, c, b):
        return [pltpu.make_async_copy(te0p.at[st_a.at[r]], accs[b], gsems[b]),
                pltpu.make_async_copy(te1.at[st_b.at[r]], bbs[b], gsems[b])]

    def ev_compute(s, b):
        a, bbuf = accs[b], bbs[b]

        @plsc.parallel_loop(0, B, unroll=4)
        def _(j):
            for k in range(4):
                sl = pl.ds(k * 16, 16)
                plsc.addupdate(a.at[j, sl], bbuf[j, sl])

    run_pipelined(CPW_EV, C_EV, c_lo, cl, ev_gathers, ev_compute)
    _scope_ev.__exit__(None, None, None)


N_PAD = 3907 * B

_sc_scatter = functools.partial(
    pl.kernel,
    out_type=jax.ShapeDtypeStruct((N_PAD, HID), jnp.float32),
    mesh=_MESH,
    scratch_types=[
        pltpu.VMEM((S_NOTE, B), jnp.int32),
        pltpu.VMEM((S_DEV, B), jnp.int32),
        pltpu.VMEM((S_DEV, B), jnp.int32),
        pltpu.VMEM((S_DEV, B), jnp.int32),
        pltpu.VMEM((B, HID), jnp.float32),
        pltpu.VMEM((B, HID), jnp.float32),
        pltpu.VMEM((B, HID), jnp.float32),
        pltpu.VMEM((B, HID), jnp.float32),
        pltpu.VMEM((B, HID), jnp.float32),
        pltpu.VMEM((B, HID), jnp.float32),
        pltpu.VMEM((B, HID), jnp.float32),
        pltpu.VMEM((B, HID), jnp.float32),
        pltpu.VMEM((B, HID), jnp.float32),
        pltpu.VMEM((B, HID), jnp.float32),
        pltpu.VMEM((S_USER * B,), jnp.float32),
        pltpu.VMEM((S_USER * B,), jnp.float32),
        pltpu.VMEM((2, HID), jnp.float32),
        pltpu.VMEM((1, HID), jnp.float32),
        pltpu.VMEM((5, HID), jnp.float32),
        pltpu.SemaphoreType.DMA,
        pltpu.SemaphoreType.DMA,
        pltpu.SemaphoreType.DMA,
        pltpu.SemaphoreType.DMA,
        pltpu.SemaphoreType.DMA,
        pltpu.SemaphoreType.DMA,
    ],
    compiler_params=pltpu.CompilerParams(use_tc_tiling_on_sc=False),
)(_sc_body)


NCH = N_PAD // B
CPW_T = _cdiv(NCH, NW)


def _tr_body(src, t4, sb0, sb1, sb2, db0, db1, db2,
             rs0, rs1, rs2, ws0, ws1, ws2):
    wid = lax.axis_index("s") * 2 + lax.axis_index("c")
    sbs = (sb0, sb1, sb2)
    dbs = (db0, db1, db2)
    rsems = (rs0, rs1, rs2)
    wsems = (ws0, ws1, ws2)
    c_lo = wid * CPW_T
    base_iota = lax.iota(jnp.int32, 16)

    def pred(s):
        return (s >= 0) & (s < CPW_T) & ((c_lo + s) < NCH)

    rowvs = [base_iota + m * 16 for m in range(8)]

    def mk_read(c, b):
        return pltpu.make_async_copy(src.at[pl.ds(c * B, B)],
                                     sbs[b].at[:, pl.ds(0, HID)], rsems[b])

    def mk_write(c, b):
        return pltpu.make_async_copy(dbs[b], t4.at[:, c], wsems[b])

    def r_issue(s, b):
        @pl.when(pred(s))
        def _():
            mk_read(c_lo + s, b).start()

    def r_wait(s, b):
        @pl.when(pred(s))
        def _():
            mk_read(c_lo + s, b).wait()

    def w_issue(s, b):
        @pl.when(pred(s))
        def _():
            mk_write(c_lo + s, b).start()

    def w_wait(s, b):
        @pl.when(pred(s))
        def _():
            mk_write(c_lo + s, b).wait()

    def transpose(b):
        sb, db = sbs[b], dbs[b]

        @plsc.parallel_loop(0, 8, unroll=2)
        def _(i):
            for s8 in range(8):
                colv = base_iota * 0 + (i * 8 + s8)
                for m in range(8):
                    g = plsc.load_gather(sb, [rowvs[m], colv])
                    db[i, s8, pl.ds(m * 16, 16)] = g

    r_issue(jnp.int32(0), 0)
    r_issue(jnp.int32(1), 1)

    def grp(g, _):
        s0 = g * NBUF
        for bi in range(NBUF):
            s = s0 + bi
            r_wait(s, bi)

            @pl.when(pred(s))
            def _(bi=bi):
                transpose(bi)
            w_issue(s, bi)
            w_wait(s - 1, (bi - 1) % NBUF)
            r_issue(s + 2, (bi + 2) % NBUF)
        return 0
    lax.fori_loop(0, _cdiv(CPW_T, NBUF), grp, 0)
    w_wait(_cdiv(CPW_T, NBUF) * NBUF - 1, (NBUF - 1) % NBUF)


_sc_transpose = functools.partial(
    pl.kernel,
    out_type=jax.ShapeDtypeStruct((8, NCH, 8, B), jnp.float32),
    mesh=_MESH,
    scratch_types=[
        pltpu.VMEM((B, HID + 1), jnp.float32),
        pltpu.VMEM((B, HID + 1), jnp.float32),
        pltpu.VMEM((B, HID + 1), jnp.float32),
        pltpu.VMEM((8, 8, B), jnp.float32),
        pltpu.VMEM((8, 8, B), jnp.float32),
        pltpu.VMEM((8, 8, B), jnp.float32),
        pltpu.SemaphoreType.DMA,
        pltpu.SemaphoreType.DMA,
        pltpu.SemaphoreType.DMA,
        pltpu.SemaphoreType.DMA,
        pltpu.SemaphoreType.DMA,
        pltpu.SemaphoreType.DMA,
    ],
    compiler_params=pltpu.CompilerParams(use_tc_tiling_on_sc=False,
                                         needs_layout_passes=False),
)(_tr_body)


def _pad_chunks(x, nc):
    return jnp.pad(x, (0, nc * B - x.shape[0]), mode='edge').reshape(nc, B)


def kernel(device_node_id, ip_node_id, user_node_id, note_node_id,
           event_node_id, device_cat, event_cat, user_dense,
           node_type_table, dev_emb0, dev_emb1, dev_emb2, W_dev,
           ev_emb0, ev_emb1, W_ev, W_user, b_user):
    i32 = jnp.int32
    d_nid = _pad_chunks(device_node_id.astype(i32), C_DEV)
    d_i0 = _pad_chunks(device_cat[:, 0].astype(i32), C_DEV)
    d_i1 = _pad_chunks(device_cat[:, 1].astype(i32), C_DEV)
    d_i2 = _pad_chunks(device_cat[:, 2].astype(i32), C_DEV)
    i_nid = _pad_chunks(ip_node_id.astype(i32), C_IP)
    u_nid = _pad_chunks(user_node_id.astype(i32), C_USER)
    n_nid = _pad_chunks(note_node_id.astype(i32), C_NOTE)
    e_nid = _pad_chunks(event_node_id.astype(i32), C_EV)
    e_i0 = _pad_chunks(event_cat[:, 0].astype(i32), C_EV)
    e_i1 = _pad_chunks(event_cat[:, 1].astype(i32), C_EV)

    t0p, t1, t2, te0p, te1 = _tc_tables(
        dev_emb0, dev_emb1, dev_emb2, W_dev, ev_emb0, ev_emb1, W_ev,
        node_type_table)

    npad = C_USER * B - N_USER
    ud0 = jnp.pad(user_dense[:, 0], (0, npad), mode='edge')
    ud1 = jnp.pad(user_dense[:, 1], (0, npad), mode='edge')

    dense = _sc_scatter(d_nid, d_i0, d_i1, d_i2, i_nid, u_nid, n_nid,
                        e_nid, e_i0, e_i1, t0p, t1, t2, te0p, te1,
                        ud0, ud1, W_user, b_user.reshape(1, HID),
                        node_type_table)
    t4 = _sc_transpose(dense)
    return t4.transpose(1, 3, 0, 2).reshape(N_PAD, HID)[:N_TOTAL]


# speedup vs baseline: 1.4426x; 1.1089x over previous
import functools

import jax
import jax.numpy as jnp
from jax import lax
from jax.experimental import pallas as pl
from jax.experimental.pallas import tpu as pltpu
from jax.experimental.pallas import tpu_sc as plsc

HID = 64
B = 128
NW = 32
NBUF = 3

N_DEV = 100000
N_IP = 100000
N_USER = 100000
N_NOTE = 150000
N_EV = 50000
N_TOTAL = 500000


def _cdiv(a, b):
    return (a + b - 1) // b


def _pad8(n):
    return _cdiv(n, 8) * 8


C_DEV = _pad8(_cdiv(N_DEV, B))
C_IP = _pad8(_cdiv(N_IP, B))
C_USER = _pad8(_cdiv(N_USER, B))
C_NOTE = _pad8(_cdiv(N_NOTE, B))
C_EV = _pad8(_cdiv(N_EV, B))

CPW_DEV = _cdiv(C_DEV, NW)
CPW_IP = _cdiv(C_IP, NW)
CPW_USER = _cdiv(C_USER, NW)
CPW_NOTE = _cdiv(C_NOTE, NW)
CPW_EV = _cdiv(C_EV, NW)


def _stage_rows(cpw):
    return (_cdiv(cpw, 8) + 1) * 8


S_DEV = _stage_rows(CPW_DEV)
S_IP = _stage_rows(CPW_IP)
S_USER = _stage_rows(CPW_USER)
S_NOTE = _stage_rows(CPW_NOTE)
S_EV = _stage_rows(CPW_EV)


def _tables_body(de0, de1, de2, wdev, ee0, ee1, wev, ntt,
                 t0p, t1, t2, te0p, te1):
    wd = wdev[...]
    we = wev[...]
    n = ntt[...]
    t0p[...] = jnp.dot(de0[...], wd[0:16, :],
                       preferred_element_type=jnp.float32) + n[0:1, :]
    t1[...] = jnp.dot(de1[...], wd[16:24, :],
                      preferred_element_type=jnp.float32)
    t2[...] = jnp.dot(de2[...], wd[24:40, :],
                      preferred_element_type=jnp.float32)
    te0p[...] = jnp.dot(ee0[...], we[0:16, :],
                        preferred_element_type=jnp.float32) + n[4:5, :]
    te1[...] = jnp.dot(ee1[...], we[16:24, :],
                       preferred_element_type=jnp.float32)


def _tc_tables(de0, de1, de2, wdev, ee0, ee1, wev, ntt):
    return pl.pallas_call(
        _tables_body,
        out_shape=[
            jax.ShapeDtypeStruct((1000, HID), jnp.float32),
            jax.ShapeDtypeStruct((50, HID), jnp.float32),
            jax.ShapeDtypeStruct((20000, HID), jnp.float32),
            jax.ShapeDtypeStruct((500, HID), jnp.float32),
            jax.ShapeDtypeStruct((100, HID), jnp.float32),
        ],
    )(de0, de1, de2, wdev, ee0, ee1, wev, ntt)


_MESH = plsc.VectorSubcoreMesh(
    core_axis_name="c", subcore_axis_name="s", num_cores=2, num_subcores=16)


def _sc_body(d_nid, d_i0, d_i1, d_i2, i_nid, u_nid, n_nid, e_nid, e_i0, e_i1,
             t0p, t1, t2, te0p, te1, ud0, ud1, wu, bu, ntt, out,
             st_nid, st_a, st_b, st_c,
             acc0, acc1, acc2, bb0, bb1, bb2, cc0, cc1, cc2,
             cbuf, udv0, udv1, wuv, buv, nttv,
             gs0, gs1, gs2, ss0, ss1, ss2):
    wid = lax.axis_index("s") * 2 + lax.axis_index("c")
    accs = (acc0, acc1, acc2)
    bbs = (bb0, bb1, bb2)
    ccs = (cc0, cc1, cc2)
    gsems = (gs0, gs1, gs2)
    ssems = (ss0, ss1, ss2)

    def ranges(cpw, s, nc, rot):
        ws = lax.rem(wid + rot, NW)
        c_lo = ws * cpw
        cl = jnp.minimum((c_lo // 8) * 8, nc - s)
        return c_lo, pl.multiple_of(cl, 8)

    def run_pipelined(cpw, nc, c_lo, cl, mk_gathers, compute=None):
        ngrp = _cdiv(cpw, NBUF)

        def pred(s):
            return (s >= 0) & (s < cpw) & ((c_lo + s) < nc)

        def mk_scatter(r, b):
            return pltpu.make_async_copy(accs[b], out.at[st_nid.at[r]],
                                         ssems[b])

        def g_issue(s, b):
            @pl.when(pred(s))
            def _():
                c = c_lo + s
                for d in mk_gathers(c - cl, c, b):
                    d.start()

        def g_wait(s, b):
            @pl.when(pred(s))
            def _():
                c = c_lo + s
                for d in mk_gathers(c - cl, c, b):
                    d.wait()

        def s_issue(s, b):
            @pl.when(pred(s))
            def _():
                mk_scatter(c_lo + s - cl, b).start()

        def s_wait(s, b):
            @pl.when(pred(s))
            def _():
                mk_scatter(c_lo + s - cl, b).wait()

        g_issue(jnp.int32(0), 0)
        g_issue(jnp.int32(1), 1)

        def grp(g, _):
            s0 = g * NBUF
            for bi in range(NBUF):
                s = s0 + bi
                g_wait(s, bi)
                if compute is not None:
                    @pl.when(pred(s))
                    def _(s=s, bi=bi):
                        compute(s, bi)
                s_issue(s, bi)
                s_wait(s - 1, (bi - 1) % NBUF)
                g_issue(s + 2, (bi + 2) % NBUF)
            return 0
        lax.fori_loop(0, ngrp, grp, 0)
        s_wait(ngrp * NBUF - 1, (NBUF - 1) % NBUF)

    def run_const(cpw, nc, c_lo, cl):
        def issue(t, _):
            c = c_lo + t

            @pl.when(c < nc)
            def _():
                pltpu.async_copy(cbuf, out.at[st_nid.at[c - cl]], ss0)
            return 0
        lax.fori_loop(0, cpw, issue, 0)

        def drain(t, _):
            c = c_lo + t

            @pl.when(c < nc)
            def _():
                pltpu.make_async_copy(cbuf, out.at[st_nid.at[c - cl]],
                                      ss0).wait()
            return 0
        lax.fori_loop(0, cpw, drain, 0)

    def fill_cbuf(type_row):
        @plsc.parallel_loop(0, B, unroll=4)
        def _(j):
            for k in range(4):
                sl = pl.ds(k * 16, 16)
                cbuf[j, sl] = nttv[type_row, sl]

    pltpu.sync_copy(ntt, nttv)
    pltpu.sync_copy(wu, wuv)
    pltpu.sync_copy(bu, buv)

    _scope_dev = jax.named_scope("seg_dev"); _scope_dev.__enter__()
    c_lo, cl = ranges(CPW_DEV, S_DEV, C_DEV, 0)
    pltpu.sync_copy(d_nid.at[pl.ds(cl, S_DEV)], st_nid.at[pl.ds(0, S_DEV)])
    pltpu.sync_copy(d_i0.at[pl.ds(cl, S_DEV)], st_a)
    pltpu.sync_copy(d_i1.at[pl.ds(cl, S_DEV)], st_b)
    pltpu.sync_copy(d_i2.at[pl.ds(cl, S_DEV)], st_c)

    def dev_gathers(r, c, b):
        return [pltpu.make_async_copy(t0p.at[st_a.at[r]], accs[b], gsems[b]),
                pltpu.make_async_copy(t1.at[st_b.at[r]], bbs[b], gsems[b]),
                pltpu.make_async_copy(t2.at[st_c.at[r]], ccs[b], gsems[b])]

    def dev_compute(s, b):
        a, bbuf, cbuf2 = accs[b], bbs[b], ccs[b]

        @plsc.parallel_loop(0, B, unroll=4)
        def _(j):
            for k in range(4):
                sl = pl.ds(k * 16, 16)
                plsc.addupdate(a.at[j, sl], bbuf[j, sl] + cbuf2[j, sl])

    run_pipelined(CPW_DEV, C_DEV, c_lo, cl, dev_gathers, dev_compute)

    _scope_dev.__exit__(None, None, None)
    _scope_ip = jax.named_scope("seg_ip"); _scope_ip.__enter__()
    c_lo, cl = ranges(CPW_IP, S_IP, C_IP, 7)
    fill_cbuf(1)
    pltpu.sync_copy(i_nid.at[pl.ds(cl, S_IP)], st_nid.at[pl.ds(0, S_IP)])
    run_const(CPW_IP, C_IP, c_lo, cl)

    _scope_ip.__exit__(None, None, None)
    _scope_us = jax.named_scope("seg_user"); _scope_us.__enter__()
    c_lo, cl = ranges(CPW_USER, S_USER, C_USER, 13)
    pltpu.sync_copy(u_nid.at[pl.ds(cl, S_USER)], st_nid.at[pl.ds(0, S_USER)])
    pltpu.sync_copy(ud0.at[pl.ds(cl * B, S_USER * B)], udv0)
    pltpu.sync_copy(ud1.at[pl.ds(cl * B, S_USER * B)], udv1)

    w0s = [wuv[0, pl.ds(k * 16, 16)] for k in range(4)]
    w1s = [wuv[1, pl.ds(k * 16, 16)] for k in range(4)]
    bs = [buv[0, pl.ds(k * 16, 16)] for k in range(4)]

    def user_compute(s, b):
        a = accs[b]
        base = (c_lo + s - cl) * B

        @plsc.parallel_loop(0, B // 16, unroll=2)
        def _(m):
            u0v = udv0[pl.ds(base + m * 16, 16)]
            u1v = udv1[pl.ds(base + m * 16, 16)]
            for jj in range(16):
                j = m * 16 + jj
                for k in range(4):
                    sl = pl.ds(k * 16, 16)
                    a[j, sl] = u0v[jj] * w0s[k] + u1v[jj] * w1s[k] + bs[k]

    run_pipelined(CPW_USER, C_USER, c_lo, cl, lambda r, c, b: [],
                  user_compute)

    _scope_us.__exit__(None, None, None)
    _scope_nt = jax.named_scope("seg_note"); _scope_nt.__enter__()
    c_lo, cl = ranges(CPW_NOTE, S_NOTE, C_NOTE, 19)
    fill_cbuf(3)
    pltpu.sync_copy(n_nid.at[pl.ds(cl, S_NOTE)], st_nid)
    run_const(CPW_NOTE, C_NOTE, c_lo, cl)

    _scope_nt.__exit__(None, None, None)
    _scope_ev = jax.named_scope("seg_ev"); _scope_ev.__enter__()
    c_lo, cl = ranges(CPW_EV, S_EV, C_EV, 26)
    pltpu.sync_copy(e_nid.at[pl.ds(cl, S_EV)], st_nid.at[pl.ds(0, S_EV)])
    pltpu.sync_copy(e_i0.at[pl.ds(cl, S_EV)], st_a.at[pl.ds(0, S_EV)])
    pltpu.sync_copy(e_i1.at[pl.ds(cl, S_EV)], st_b.at[pl.ds(0, S_EV)])

    def ev_gathers(r, c, b):
        return [pltpu.make_async_copy(te0p.at[st_a.at[r]], accs[b], gsems[b]),
                pltpu.make_async_copy(te1.at[st_b.at[r]], bbs[b], gsems[b])]

    def ev_compute(s, b):
        a, bbuf = accs[b], bbs[b]

        @plsc.parallel_loop(0, B, unroll=4)
        def _(j):
            for k in range(4):
                sl = pl.ds(k * 16, 16)
                plsc.addupdate(a.at[j, sl], bbuf[j, sl])

    run_pipelined(CPW_EV, C_EV, c_lo, cl, ev_gathers, ev_compute)
    _scope_ev.__exit__(None, None, None)


N_PAD = 3907 * B

_sc_scatter = functools.partial(
    pl.kernel,
    out_type=jax.ShapeDtypeStruct((N_PAD, HID), jnp.float32),
    mesh=_MESH,
    scratch_types=[
        pltpu.VMEM((S_NOTE, B), jnp.int32),
        pltpu.VMEM((S_DEV, B), jnp.int32),
        pltpu.VMEM((S_DEV, B), jnp.int32),
        pltpu.VMEM((S_DEV, B), jnp.int32),
        pltpu.VMEM((B, HID), jnp.float32),
        pltpu.VMEM((B, HID), jnp.float32),
        pltpu.VMEM((B, HID), jnp.float32),
        pltpu.VMEM((B, HID), jnp.float32),
        pltpu.VMEM((B, HID), jnp.float32),
        pltpu.VMEM((B, HID), jnp.float32),
        pltpu.VMEM((B, HID), jnp.float32),
        pltpu.VMEM((B, HID), jnp.float32),
        pltpu.VMEM((B, HID), jnp.float32),
        pltpu.VMEM((B, HID), jnp.float32),
        pltpu.VMEM((S_USER * B,), jnp.float32),
        pltpu.VMEM((S_USER * B,), jnp.float32),
        pltpu.VMEM((2, HID), jnp.float32),
        pltpu.VMEM((1, HID), jnp.float32),
        pltpu.VMEM((5, HID), jnp.float32),
        pltpu.SemaphoreType.DMA,
        pltpu.SemaphoreType.DMA,
        pltpu.SemaphoreType.DMA,
        pltpu.SemaphoreType.DMA,
        pltpu.SemaphoreType.DMA,
        pltpu.SemaphoreType.DMA,
    ],
    compiler_params=pltpu.CompilerParams(use_tc_tiling_on_sc=False),
)(_sc_body)


NCH = N_PAD // B
CPW_T = _cdiv(NCH, NW)


def _tr_body(src, t4, sb0, sb1, sb2, db0, db1, db2,
             rs0, rs1, rs2, ws0, ws1, ws2):
    wid = lax.axis_index("s") * 2 + lax.axis_index("c")
    sbs = (sb0, sb1, sb2)
    dbs = (db0, db1, db2)
    rsems = (rs0, rs1, rs2)
    wsems = (ws0, ws1, ws2)
    c_lo = wid * CPW_T
    base_iota = lax.iota(jnp.int32, 16)

    def pred(s):
        return (s >= 0) & (s < CPW_T) & ((c_lo + s) < NCH)

    rowvs = [base_iota + m * 16 for m in range(8)]

    def mk_read(c, b):
        return pltpu.make_async_copy(src.at[pl.ds(c * B, B)],
                                     sbs[b].at[:, pl.ds(0, HID)], rsems[b])

    def mk_write(c, b):
        return pltpu.make_async_copy(dbs[b], t4.at[:, c], wsems[b])

    def r_issue(s, b):
        @pl.when(pred(s))
        def _():
            mk_read(c_lo + s, b).start()

    def r_wait(s, b):
        @pl.when(pred(s))
        def _():
            mk_read(c_lo + s, b).wait()

    def w_issue(s, b):
        @pl.when(pred(s))
        def _():
            mk_write(c_lo + s, b).start()

    def w_wait(s, b):
        @pl.when(pred(s))
        def _():
            mk_write(c_lo + s, b).wait()

    def transpose(b):
        sb, db = sbs[b], dbs[b]

        @plsc.parallel_loop(0, 8, unroll=2)
        def _(i):
            for s8 in range(8):
                colv = base_iota * 0 + (i * 8 + s8)
                for m in range(8):
                    g = plsc.load_gather(sb, [rowvs[m], colv])
                    db[i, s8, pl.ds(m * 16, 16)] = g

    r_issue(jnp.int32(0), 0)
    r_issue(jnp.int32(1), 1)

    def grp(g, _):
        s0 = g * NBUF
        for bi in range(NBUF):
            s = s0 + bi
            r_wait(s, bi)

            @pl.when(pred(s))
            def _(bi=bi):
                transpose(bi)
            w_issue(s, bi)
            w_wait(s - 1, (bi - 1) % NBUF)
            r_issue(s + 2, (bi + 2) % NBUF)
        return 0
    lax.fori_loop(0, _cdiv(CPW_T, NBUF), grp, 0)
    w_wait(_cdiv(CPW_T, NBUF) * NBUF - 1, (NBUF - 1) % NBUF)


_sc_transpose = functools.partial(
    pl.kernel,
    out_type=jax.ShapeDtypeStruct((8, NCH, 8, B), jnp.float32),
    mesh=_MESH,
    scratch_types=[
        pltpu.VMEM((B, HID + 1), jnp.float32),
        pltpu.VMEM((B, HID + 1), jnp.float32),
        pltpu.VMEM((B, HID + 1), jnp.float32),
        pltpu.VMEM((8, 8, B), jnp.float32),
        pltpu.VMEM((8, 8, B), jnp.float32),
        pltpu.VMEM((8, 8, B), jnp.float32),
        pltpu.SemaphoreType.DMA,
        pltpu.SemaphoreType.DMA,
        pltpu.SemaphoreType.DMA,
        pltpu.SemaphoreType.DMA,
        pltpu.SemaphoreType.DMA,
        pltpu.SemaphoreType.DMA,
    ],
    compiler_params=pltpu.CompilerParams(use_tc_tiling_on_sc=False,
                                         needs_layout_passes=False),
)(_tr_body)


def _pad_chunks(x, nc):
    return jnp.pad(x, (0, nc * B - x.shape[0]), mode='edge').reshape(nc, B)


def kernel(device_node_id, ip_node_id, user_node_id, note_node_id,
           event_node_id, device_cat, event_cat, user_dense,
           node_type_table, dev_emb0, dev_emb1, dev_emb2, W_dev,
           ev_emb0, ev_emb1, W_ev, W_user, b_user):
    i32 = jnp.int32
    d_nid = _pad_chunks(device_node_id.astype(i32), C_DEV)
    d_i0 = _pad_chunks(device_cat[:, 0].astype(i32), C_DEV)
    d_i1 = _pad_chunks(device_cat[:, 1].astype(i32), C_DEV)
    d_i2 = _pad_chunks(device_cat[:, 2].astype(i32), C_DEV)
    i_nid = _pad_chunks(ip_node_id.astype(i32), C_IP)
    u_nid = _pad_chunks(user_node_id.astype(i32), C_USER)
    n_nid = _pad_chunks(note_node_id.astype(i32), C_NOTE)
    e_nid = _pad_chunks(event_node_id.astype(i32), C_EV)
    e_i0 = _pad_chunks(event_cat[:, 0].astype(i32), C_EV)
    e_i1 = _pad_chunks(event_cat[:, 1].astype(i32), C_EV)

    t0p, t1, t2, te0p, te1 = _tc_tables(
        dev_emb0, dev_emb1, dev_emb2, W_dev, ev_emb0, ev_emb1, W_ev,
        node_type_table)

    npad = C_USER * B - N_USER
    ud0 = jnp.pad(user_dense[:, 0], (0, npad), mode='edge')
    ud1 = jnp.pad(user_dense[:, 1], (0, npad), mode='edge')

    dense = _sc_scatter(d_nid, d_i0, d_i1, d_i2, i_nid, u_nid, n_nid,
                        e_nid, e_i0, e_i1, t0p, t1, t2, te0p, te1,
                        ud0, ud1, W_user, b_user.reshape(1, HID),
                        node_type_table)
    t4 = _sc_transpose(dense)
    return t4.transpose(1, 3, 0, 2).reshape(N_PAD, HID)[:N_TOTAL]
